# baseline (speedup 1.0000x reference)
import jax
import jax.numpy as jnp
from jax.experimental import pallas as pl
from jax.experimental.pallas import tpu as pltpu

_B = 16
_C = 85
_N3, _N4, _N5 = 4096, 1024, 256
_NTOT = _N3 + _N4 + _N5
_R, _L = 42, 128
_K = 50
_NGT = 200
_EPS = 1e-7

_ATAN_C = (0.9999999581953061, -0.3333230282771013, 0.19973681363449028,
           -0.14040138891201454, 0.09967923618944668, -0.060219127990167355,
           0.024756780690475755, -0.00483116838738874)
_HALF_PI = 1.5707963267948966


def _atan(x):
    r = jnp.abs(x)
    z = jnp.minimum(r, 1.0 / r)
    t = z * z
    p = jnp.float32(_ATAN_C[7])
    for c in _ATAN_C[6::-1]:
        p = p * t + jnp.float32(c)
    p = z * p
    res = jnp.where(r <= 1.0, p, _HALF_PI - p)
    return jnp.where(x < 0, -res, res)


def _topk_ranks(o_ref):
    obj = o_ref[:, 4].reshape(_B, _R, _L)
    bits = jax.lax.bitcast_convert_type(obj, jnp.int32)
    skey = bits ^ (jax.lax.shift_right_arithmetic(bits, 31)
                   & jnp.int32(0x7FFFFFFF))

    def bs_body(i, lohi):
        lo, hi = lohi
        mid = (lo >> 1) + (hi >> 1) + ((lo | hi) & 1)
        ge = (skey >= mid).astype(jnp.float32)
        cnt = jnp.sum(jnp.sum(ge, axis=2, keepdims=True),
                      axis=1, keepdims=True)
        take = cnt >= float(_K)
        return jnp.where(take, mid, lo), jnp.where(take, hi, mid - 1)

    lo0 = jnp.full((_B, 1, 1), jnp.int32(-2147483648))
    hi0 = jnp.full((_B, 1, 1), jnp.int32(2147483647))
    thr, _ = jax.lax.fori_loop(0, 32, bs_body, (lo0, hi0), unroll=False)

    lt_l = (jax.lax.broadcasted_iota(jnp.int32, (_L, _L), 0)
            < jax.lax.broadcasted_iota(jnp.int32, (_L, _L), 1)).astype(jnp.float32)
    lt_r = (jax.lax.broadcasted_iota(jnp.int32, (_R, _R), 0)
            < jax.lax.broadcasted_iota(jnp.int32, (_R, _R), 1)).astype(jnp.float32)
    dn2 = (((1,), (0,)), ((), ()))

    def eprefix(mf):
        lane_pref = jax.lax.dot_general(
            mf.reshape(_B * _R, _L), lt_l, dn2,
            preferred_element_type=jnp.float32).reshape(_B, _R, _L)
        rowsum = jnp.sum(mf, axis=2)
        roff = jax.lax.dot_general(rowsum, lt_r, dn2,
                                   preferred_element_type=jnp.float32)
        return lane_pref + roff.reshape(_B, _R, 1)

    strict = (skey > thr).astype(jnp.float32)
    ties = (skey == thr).astype(jnp.float32)
    n1 = jnp.sum(jnp.sum(strict, axis=2, keepdims=True),
                 axis=1, keepdims=True)
    msel = jnp.maximum(strict,
                       ties * (eprefix(ties) < (float(_K) - n1)))
    cc = jnp.where(msel > 0.0, eprefix(msel), -1.0)
    return cc.reshape(_B, _NTOT)


def _loss_kernel(o_ref, p_ref, gt_ref, bidx_ref, gcls_ref,
                 out_ref, cc_scr):
    b = pl.program_id(0)

    @pl.when(b == 0)
    def _init():
        out_ref[...] = jnp.zeros_like(out_ref)
        cc_scr[...] = _topk_ranks(o_ref)

    cc_flat = cc_scr[pl.ds(b, 1), :]
    kcol = jax.lax.broadcasted_iota(jnp.int32, (_K, 1), 0).astype(jnp.float32)
    s = (cc_flat == kcol).astype(jnp.float32)
    dn = (((1,), (1,)), ((), ()))
    sel = jax.lax.dot_general(s, p_ref[0], dn,
                              preferred_element_type=jnp.float32)

    ch = jax.lax.broadcasted_iota(jnp.int32, (1, _C), 1)

    def pick(c):
        return jnp.sum(jnp.where(ch == c, sel, 0.0), axis=1, keepdims=True)

    b1x, b1y, b1w, b1h, so = pick(0), pick(1), pick(2), pick(3), pick(4)

    gx = gt_ref[0:1, :]
    gy = gt_ref[1:2, :]
    gw = gt_ref[2:3, :]
    gh = gt_ref[3:4, :]

    b1x1, b1y1 = b1x - b1w * 0.5, b1y - b1h * 0.5
    b1x2, b1y2 = b1x + b1w * 0.5, b1y + b1h * 0.5
    b2x1, b2y1 = gx - gw * 0.5, gy - gh * 0.5
    b2x2, b2y2 = gx + gw * 0.5, gy + gh * 0.5
    iw = jnp.clip(jnp.minimum(b1x2, b2x2) - jnp.maximum(b1x1, b2x1), 0.0, None)
    ih = jnp.clip(jnp.minimum(b1y2, b2y2) - jnp.maximum(b1y1, b2y1), 0.0, None)
    inter = iw * ih
    union = b1w * b1h + gw * gh - inter + _EPS
    iou = inter / union
    cw = jnp.maximum(b1x2, b2x2) - jnp.minimum(b1x1, b2x1)
    chh = jnp.maximum(b1y2, b2y2) - jnp.minimum(b1y1, b2y1)
    c2 = cw * cw + chh * chh + _EPS
    rho2 = (gx - b1x) ** 2 + (gy - b1y) ** 2
    at1 = _atan(b1w / b1h)
    at2 = _atan(gw / gh)
    v = (4.0 / (3.141592653589793 ** 2)) * (at2 - at1) ** 2
    alpha = v / (1.0 - iou + v + _EPS)
    ciou = jnp.clip(iou - (rho2 / c2 + v * alpha), 0.0, 1.0)

    bidx = bidx_ref[...]
    mask = bidx == b
    cm = jnp.where(mask, ciou, -1.0)
    cmax = jnp.max(cm, axis=1, keepdims=True)
    jiota = jax.lax.broadcasted_iota(jnp.int32, (1, _NGT), 1)
    eqm = cm == cmax
    midx = jnp.min(jnp.where(eqm, jiota, _NGT), axis=1, keepdims=True)
    gsel = jnp.sum(jnp.where(jiota == midx, gcls_ref[...], 0.0),
                   axis=1, keepdims=True)

    box_loss = jnp.mean(1.0 - cmax)

    def bce(x, t):
        return jnp.maximum(x, 0.0) - x * t + jnp.log1p(jnp.exp(-jnp.abs(x)))

    obj_loss = jnp.mean(bce(so, cmax))

    chf = ch.astype(jnp.float32)
    cls_mask = ch >= 5
    tgt = jnp.where((chf - 5.0) == gsel, 1.0, 0.0)
    fcls = bce(sel, tgt)
    cls_loss = jnp.sum(jnp.where(cls_mask, fcls, 0.0)) / (_K * (_C - 5))

    has_any = jnp.any(mask)
    l128 = jax.lax.broadcasted_iota(jnp.int32, (1, 128), 1)
    vec = jnp.where(l128 == 0, box_loss,
                    jnp.where(l128 == 1, obj_loss,
                              jnp.where(l128 == 2, cls_loss, 0.0)))
    out_ref[...] += jnp.where(has_any, vec, 0.0)


def kernel(p3, p4, p5, bboxes, cls, batch_idx):
    pall = jnp.concatenate(
        [p3.reshape(_B, _C, _N3), p4.reshape(_B, _C, _N4),
         p5.reshape(_B, _C, _N5)], axis=2)
    gt_t = bboxes.T
    bidx = batch_idx.astype(jnp.int32).reshape(1, _NGT)
    gcls = cls[:, 0].astype(jnp.float32).reshape(1, _NGT)

    def fixed(b):
        return (0, 0)

    out = pl.pallas_call(
        _loss_kernel,
        grid=(_B,),
        in_specs=[
            pl.BlockSpec((_B, 8, _NTOT), lambda b: (0, 0, 0)),
            pl.BlockSpec((1, _C, _NTOT), lambda b: (b, 0, 0)),
            pl.BlockSpec((4, _NGT), fixed),
            pl.BlockSpec((1, _NGT), fixed),
            pl.BlockSpec((1, _NGT), fixed),
        ],
        out_specs=pl.BlockSpec((1, 128), lambda b: (0, 0)),
        out_shape=jax.ShapeDtypeStruct((1, 128), jnp.float32),
        scratch_shapes=[pltpu.VMEM((_B, _NTOT), jnp.float32)],
        compiler_params=pltpu.CompilerParams(
            dimension_semantics=("arbitrary",)),
    )(pall, pall, gt_t, bidx, gcls)

    lb = out[0, 0] / _B
    lo = out[0, 1] / _B
    lc = out[0, 2] / _B
    total = 0.05 * lb + 1.0 * lo + 0.5 * lc
    return (total, lb, lo, lc)


# speedup vs baseline: 1.2425x; 1.2425x over previous
import jax
import jax.numpy as jnp
from jax.experimental import pallas as pl
from jax.experimental.pallas import tpu as pltpu

_B = 16
_C = 85
_N3, _N4, _N5 = 4096, 1024, 256
_NTOT = _N3 + _N4 + _N5
_R, _L = 42, 128
_K = 50
_NGT = 200
_EPS = 1e-7

_ATAN_C = (0.9999999581953061, -0.3333230282771013, 0.19973681363449028,
           -0.14040138891201454, 0.09967923618944668, -0.060219127990167355,
           0.024756780690475755, -0.00483116838738874)
_HALF_PI = 1.5707963267948966


def _atan(x):
    r = jnp.abs(x)
    z = jnp.minimum(r, 1.0 / r)
    t = z * z
    p = jnp.float32(_ATAN_C[7])
    for c in _ATAN_C[6::-1]:
        p = p * t + jnp.float32(c)
    p = z * p
    res = jnp.where(r <= 1.0, p, _HALF_PI - p)
    return jnp.where(x < 0, -res, res)


def _topk_ranks(o3_ref, o4_ref, o5_ref):
    obj = jnp.concatenate(
        [o3_ref[:, 4, :].reshape(_B, _N3 // _L, _L),
         o4_ref[:, 4, :].reshape(_B, _N4 // _L, _L),
         o5_ref[:, 4, :].reshape(_B, _N5 // _L, _L)], axis=1)
    bits = jax.lax.bitcast_convert_type(obj, jnp.int32)
    skey = bits ^ (jax.lax.shift_right_arithmetic(bits, 31)
                   & jnp.int32(0x7FFFFFFF))

    def bs_body(i, lohi):
        lo, hi = lohi
        mid = (lo >> 1) + (hi >> 1) + ((lo | hi) & 1)
        ge = (skey >= mid).astype(jnp.float32)
        cnt = jnp.sum(jnp.sum(ge, axis=2, keepdims=True),
                      axis=1, keepdims=True)
        take = cnt >= float(_K)
        return jnp.where(take, mid, lo), jnp.where(take, hi, mid - 1)

    lo0 = jnp.full((_B, 1, 1), jnp.int32(-2147483648))
    hi0 = jnp.full((_B, 1, 1), jnp.int32(2147483647))
    thr, _ = jax.lax.fori_loop(0, 32, bs_body, (lo0, hi0), unroll=False)

    lt_l = (jax.lax.broadcasted_iota(jnp.int32, (_L, _L), 0)
            < jax.lax.broadcasted_iota(jnp.int32, (_L, _L), 1)).astype(jnp.float32)
    lt_r = (jax.lax.broadcasted_iota(jnp.int32, (_R, _R), 0)
            < jax.lax.broadcasted_iota(jnp.int32, (_R, _R), 1)).astype(jnp.float32)
    dn2 = (((1,), (0,)), ((), ()))

    def eprefix(mf):
        lane_pref = jax.lax.dot_general(
            mf.reshape(_B * _R, _L), lt_l, dn2,
            preferred_element_type=jnp.float32).reshape(_B, _R, _L)
        rowsum = jnp.sum(mf, axis=2)
        roff = jax.lax.dot_general(rowsum, lt_r, dn2,
                                   preferred_element_type=jnp.float32)
        return lane_pref + roff.reshape(_B, _R, 1)

    strict = (skey > thr).astype(jnp.float32)
    ties = (skey == thr).astype(jnp.float32)
    n1 = jnp.sum(jnp.sum(strict, axis=2, keepdims=True),
                 axis=1, keepdims=True)
    msel = jnp.maximum(strict,
                       ties * (eprefix(ties) < (float(_K) - n1)))
    cc = jnp.where(msel > 0.0, eprefix(msel), -1.0)
    return cc.reshape(_B, _NTOT)


def _loss_kernel(o3_ref, o4_ref, o5_ref, p3_ref, p4_ref, p5_ref,
                 gtx_ref, gty_ref, gtw_ref, gth_ref, bidx_ref, gcls_ref,
                 out_ref, cc_scr):
    b = pl.program_id(0)

    @pl.when(b == 0)
    def _init():
        out_ref[...] = jnp.zeros_like(out_ref)
        cc_scr[...] = _topk_ranks(o3_ref, o4_ref, o5_ref)

    cc_flat = cc_scr[pl.ds(b, 1), :]
    kcol = jax.lax.broadcasted_iota(jnp.int32, (_K, 1), 0).astype(jnp.float32)
    s = (cc_flat == kcol).astype(jnp.float32)
    dn = (((1,), (1,)), ((), ()))
    sel = (
        jax.lax.dot_general(s[:, :_N3], p3_ref[0], dn,
                            preferred_element_type=jnp.float32)
        + jax.lax.dot_general(s[:, _N3:_N3 + _N4], p4_ref[0], dn,
                              preferred_element_type=jnp.float32)
        + jax.lax.dot_general(s[:, _N3 + _N4:], p5_ref[0], dn,
                              preferred_element_type=jnp.float32)
    )

    ch = jax.lax.broadcasted_iota(jnp.int32, (1, _C), 1)

    def pick(c):
        return jnp.sum(jnp.where(ch == c, sel, 0.0), axis=1, keepdims=True)

    b1x, b1y, b1w, b1h, so = pick(0), pick(1), pick(2), pick(3), pick(4)

    gx, gy, gw, gh = gtx_ref[...], gty_ref[...], gtw_ref[...], gth_ref[...]

    b1x1, b1y1 = b1x - b1w * 0.5, b1y - b1h * 0.5
    b1x2, b1y2 = b1x + b1w * 0.5, b1y + b1h * 0.5
    b2x1, b2y1 = gx - gw * 0.5, gy - gh * 0.5
    b2x2, b2y2 = gx + gw * 0.5, gy + gh * 0.5
    iw = jnp.clip(jnp.minimum(b1x2, b2x2) - jnp.maximum(b1x1, b2x1), 0.0, None)
    ih = jnp.clip(jnp.minimum(b1y2, b2y2) - jnp.maximum(b1y1, b2y1), 0.0, None)
    inter = iw * ih
    union = b1w * b1h + gw * gh - inter + _EPS
    iou = inter / union
    cw = jnp.maximum(b1x2, b2x2) - jnp.minimum(b1x1, b2x1)
    chh = jnp.maximum(b1y2, b2y2) - jnp.minimum(b1y1, b2y1)
    c2 = cw * cw + chh * chh + _EPS
    rho2 = (gx - b1x) ** 2 + (gy - b1y) ** 2
    at1 = _atan(b1w / b1h)
    at2 = _atan(gw / gh)
    v = (4.0 / (3.141592653589793 ** 2)) * (at2 - at1) ** 2
    alpha = v / (1.0 - iou + v + _EPS)
    ciou = jnp.clip(iou - (rho2 / c2 + v * alpha), 0.0, 1.0)

    bidx = bidx_ref[...]
    mask = bidx == b
    cm = jnp.where(mask, ciou, -1.0)
    cmax = jnp.max(cm, axis=1, keepdims=True)
    jiota = jax.lax.broadcasted_iota(jnp.int32, (1, _NGT), 1)
    eqm = cm == cmax
    midx = jnp.min(jnp.where(eqm, jiota, _NGT), axis=1, keepdims=True)
    gsel = jnp.sum(jnp.where(jiota == midx, gcls_ref[...], 0.0),
                   axis=1, keepdims=True)

    box_loss = jnp.mean(1.0 - cmax)

    def bce(x, t):
        return jnp.maximum(x, 0.0) - x * t + jnp.log1p(jnp.exp(-jnp.abs(x)))

    obj_loss = jnp.mean(bce(so, cmax))

    chf = ch.astype(jnp.float32)
    cls_mask = ch >= 5
    tgt = jnp.where((chf - 5.0) == gsel, 1.0, 0.0)
    fcls = bce(sel, tgt)
    cls_loss = jnp.sum(jnp.where(cls_mask, fcls, 0.0)) / (_K * (_C - 5))

    has_any = jnp.any(mask)
    l128 = jax.lax.broadcasted_iota(jnp.int32, (1, 128), 1)
    vec = jnp.where(l128 == 0, box_loss,
                    jnp.where(l128 == 1, obj_loss,
                              jnp.where(l128 == 2, cls_loss, 0.0)))
    out_ref[...] += jnp.where(has_any, vec, 0.0)


def kernel(p3, p4, p5, bboxes, cls, batch_idx):
    p3f = p3.reshape(_B, _C, _N3)
    p4f = p4.reshape(_B, _C, _N4)
    p5f = p5.reshape(_B, _C, _N5)
    gtx = bboxes[:, 0].reshape(1, _NGT)
    gty = bboxes[:, 1].reshape(1, _NGT)
    gtw = bboxes[:, 2].reshape(1, _NGT)
    gth = bboxes[:, 3].reshape(1, _NGT)
    bidx = batch_idx.astype(jnp.int32).reshape(1, _NGT)
    gcls = cls[:, 0].astype(jnp.float32).reshape(1, _NGT)

    def bmap(b):
        return (b, 0, 0)

    def fixed(b):
        return (0, 0)

    out = pl.pallas_call(
        _loss_kernel,
        grid=(_B,),
        in_specs=[
            pl.BlockSpec((_B, 8, _N3), lambda b: (0, 0, 0)),
            pl.BlockSpec((_B, 8, _N4), lambda b: (0, 0, 0)),
            pl.BlockSpec((_B, 8, _N5), lambda b: (0, 0, 0)),
            pl.BlockSpec((1, _C, _N3), bmap),
            pl.BlockSpec((1, _C, _N4), bmap),
            pl.BlockSpec((1, _C, _N5), bmap),
            pl.BlockSpec((1, _NGT), fixed),
            pl.BlockSpec((1, _NGT), fixed),
            pl.BlockSpec((1, _NGT), fixed),
            pl.BlockSpec((1, _NGT), fixed),
            pl.BlockSpec((1, _NGT), fixed),
            pl.BlockSpec((1, _NGT), fixed),
        ],
        out_specs=pl.BlockSpec((1, 128), lambda b: (0, 0)),
        out_shape=jax.ShapeDtypeStruct((1, 128), jnp.float32),
        scratch_shapes=[pltpu.VMEM((_B, _NTOT), jnp.float32)],
        compiler_params=pltpu.CompilerParams(
            dimension_semantics=("arbitrary",)),
    )(p3f, p4f, p5f, p3f, p4f, p5f, gtx, gty, gtw, gth, bidx, gcls)

    lb = out[0, 0] / _B
    lo = out[0, 1] / _B
    lc = out[0, 2] / _B
    total = 0.05 * lb + 1.0 * lo + 0.5 * lc
    return (total, lb, lo, lc)


# speedup vs baseline: 1.2591x; 1.0134x over previous
import jax
import jax.numpy as jnp
from jax.experimental import pallas as pl
from jax.experimental.pallas import tpu as pltpu

_B = 16
_C = 85
_N3, _N4, _N5 = 4096, 1024, 256
_NTOT = _N3 + _N4 + _N5
_R, _L = 42, 128
_K = 50
_NGT = 200
_EPS = 1e-7

_ATAN_C = (0.9999999581953061, -0.3333230282771013, 0.19973681363449028,
           -0.14040138891201454, 0.09967923618944668, -0.060219127990167355,
           0.024756780690475755, -0.00483116838738874)
_HALF_PI = 1.5707963267948966


def _atan(x):
    r = jnp.abs(x)
    z = jnp.minimum(r, 1.0 / r)
    t = z * z
    p = jnp.float32(_ATAN_C[7])
    for c in _ATAN_C[6::-1]:
        p = p * t + jnp.float32(c)
    p = z * p
    res = jnp.where(r <= 1.0, p, _HALF_PI - p)
    return jnp.where(x < 0, -res, res)


def _topk_ranks(o3_ref, o4_ref, o5_ref):
    obj = jnp.concatenate(
        [o3_ref[:, 4, :].reshape(_B, _N3 // _L, _L),
         o4_ref[:, 4, :].reshape(_B, _N4 // _L, _L),
         o5_ref[:, 4, :].reshape(_B, _N5 // _L, _L)], axis=1)
    bits = jax.lax.bitcast_convert_type(obj, jnp.int32)
    skey = bits ^ (jax.lax.shift_right_arithmetic(bits, 31)
                   & jnp.int32(0x7FFFFFFF))

    def bs_body(i, lohi):
        lo, hi = lohi
        mid = (lo >> 1) + (hi >> 1) + ((lo | hi) & 1)
        ge = (skey >= mid).astype(jnp.float32)
        cnt = jnp.sum(jnp.sum(ge, axis=1, keepdims=True),
                      axis=2, keepdims=True)
        take = cnt >= float(_K)
        return jnp.where(take, mid, lo), jnp.where(take, hi, mid - 1)

    lo0 = jnp.full((_B, 1, 1), jnp.int32(-2147483648))
    hi0 = jnp.full((_B, 1, 1), jnp.int32(2147483647))
    thr, _ = jax.lax.fori_loop(0, 32, bs_body, (lo0, hi0), unroll=False)

    lt_l = (jax.lax.broadcasted_iota(jnp.int32, (_L, _L), 0)
            < jax.lax.broadcasted_iota(jnp.int32, (_L, _L), 1)).astype(jnp.float32)
    lt_r = (jax.lax.broadcasted_iota(jnp.int32, (_R, _R), 0)
            < jax.lax.broadcasted_iota(jnp.int32, (_R, _R), 1)).astype(jnp.float32)
    dn2 = (((1,), (0,)), ((), ()))

    def eprefix(mf):
        lane_pref = jax.lax.dot_general(
            mf.reshape(_B * _R, _L), lt_l, dn2,
            preferred_element_type=jnp.float32).reshape(_B, _R, _L)
        rowsum = jnp.sum(mf, axis=2)
        roff = jax.lax.dot_general(rowsum, lt_r, dn2,
                                   preferred_element_type=jnp.float32)
        return lane_pref + roff.reshape(_B, _R, 1)

    strict = (skey > thr).astype(jnp.float32)
    ties = (skey == thr).astype(jnp.float32)
    n1 = jnp.sum(jnp.sum(strict, axis=1, keepdims=True),
                 axis=2, keepdims=True)
    msel = jnp.maximum(strict,
                       ties * (eprefix(ties) < (float(_K) - n1)))
    cc = jnp.where(msel > 0.0, eprefix(msel), -1.0)
    return cc.reshape(_B, _NTOT)


def _loss_kernel(o3_ref, o4_ref, o5_ref, p3_ref, p4_ref, p5_ref,
                 gtx_ref, gty_ref, gtw_ref, gth_ref, bidx_ref, gcls_ref,
                 out_ref, cc_scr):
    b = pl.program_id(0)

    @pl.when(b == 0)
    def _init():
        out_ref[...] = jnp.zeros_like(out_ref)
        cc_scr[...] = _topk_ranks(o3_ref, o4_ref, o5_ref)

    cc_flat = cc_scr[pl.ds(b, 1), :]
    kcol = jax.lax.broadcasted_iota(jnp.int32, (_K, 1), 0).astype(jnp.float32)
    s = (cc_flat == kcol).astype(jnp.float32)
    dn = (((1,), (1,)), ((), ()))
    sel = (
        jax.lax.dot_general(s[:, :_N3], p3_ref[0], dn,
                            preferred_element_type=jnp.float32)
        + jax.lax.dot_general(s[:, _N3:_N3 + _N4], p4_ref[0], dn,
                              preferred_element_type=jnp.float32)
        + jax.lax.dot_general(s[:, _N3 + _N4:], p5_ref[0], dn,
                              preferred_element_type=jnp.float32)
    )

    ch = jax.lax.broadcasted_iota(jnp.int32, (1, _C), 1)

    def pick(c):
        return jnp.sum(jnp.where(ch == c, sel, 0.0), axis=1, keepdims=True)

    b1x, b1y, b1w, b1h, so = pick(0), pick(1), pick(2), pick(3), pick(4)

    gx, gy, gw, gh = gtx_ref[...], gty_ref[...], gtw_ref[...], gth_ref[...]

    b1x1, b1y1 = b1x - b1w * 0.5, b1y - b1h * 0.5
    b1x2, b1y2 = b1x + b1w * 0.5, b1y + b1h * 0.5
    b2x1, b2y1 = gx - gw * 0.5, gy - gh * 0.5
    b2x2, b2y2 = gx + gw * 0.5, gy + gh * 0.5
    iw = jnp.clip(jnp.minimum(b1x2, b2x2) - jnp.maximum(b1x1, b2x1), 0.0, None)
    ih = jnp.clip(jnp.minimum(b1y2, b2y2) - jnp.maximum(b1y1, b2y1), 0.0, None)
    inter = iw * ih
    union = b1w * b1h + gw * gh - inter + _EPS
    iou = inter / union
    cw = jnp.maximum(b1x2, b2x2) - jnp.minimum(b1x1, b2x1)
    chh = jnp.maximum(b1y2, b2y2) - jnp.minimum(b1y1, b2y1)
    c2 = cw * cw + chh * chh + _EPS
    rho2 = (gx - b1x) ** 2 + (gy - b1y) ** 2
    at1 = _atan(b1w / b1h)
    at2 = _atan(gw / gh)
    v = (4.0 / (3.141592653589793 ** 2)) * (at2 - at1) ** 2
    alpha = v / (1.0 - iou + v + _EPS)
    ciou = jnp.clip(iou - (rho2 / c2 + v * alpha), 0.0, 1.0)

    bidx = bidx_ref[...]
    mask = bidx == b
    cm = jnp.where(mask, ciou, -1.0)
    cmax = jnp.max(cm, axis=1, keepdims=True)
    jiota = jax.lax.broadcasted_iota(jnp.int32, (1, _NGT), 1)
    eqm = cm == cmax
    midx = jnp.min(jnp.where(eqm, jiota, _NGT), axis=1, keepdims=True)
    gsel = jnp.sum(jnp.where(jiota == midx, gcls_ref[...], 0.0),
                   axis=1, keepdims=True)

    box_loss = jnp.mean(1.0 - cmax)

    def bce(x, t):
        return jnp.maximum(x, 0.0) - x * t + jnp.log1p(jnp.exp(-jnp.abs(x)))

    obj_loss = jnp.mean(bce(so, cmax))

    chf = ch.astype(jnp.float32)
    cls_mask = ch >= 5
    tgt = jnp.where((chf - 5.0) == gsel, 1.0, 0.0)
    fcls = bce(sel, tgt)
    cls_loss = jnp.sum(jnp.where(cls_mask, fcls, 0.0)) / (_K * (_C - 5))

    has_any = jnp.any(mask)
    l128 = jax.lax.broadcasted_iota(jnp.int32, (1, 128), 1)
    vec = jnp.where(l128 == 0, box_loss,
                    jnp.where(l128 == 1, obj_loss,
                              jnp.where(l128 == 2, cls_loss, 0.0)))
    out_ref[...] += jnp.where(has_any, vec, 0.0)


def kernel(p3, p4, p5, bboxes, cls, batch_idx):
    p3f = p3.reshape(_B, _C, _N3)
    p4f = p4.reshape(_B, _C, _N4)
    p5f = p5.reshape(_B, _C, _N5)
    gtx = bboxes[:, 0].reshape(1, _NGT)
    gty = bboxes[:, 1].reshape(1, _NGT)
    gtw = bboxes[:, 2].reshape(1, _NGT)
    gth = bboxes[:, 3].reshape(1, _NGT)
    bidx = batch_idx.astype(jnp.int32).reshape(1, _NGT)
    gcls = cls[:, 0].astype(jnp.float32).reshape(1, _NGT)

    def bmap(b):
        return (b, 0, 0)

    def fixed(b):
        return (0, 0)

    out = pl.pallas_call(
        _loss_kernel,
        grid=(_B,),
        in_specs=[
            pl.BlockSpec((_B, 8, _N3), lambda b: (0, 0, 0)),
            pl.BlockSpec((_B, 8, _N4), lambda b: (0, 0, 0)),
            pl.BlockSpec((_B, 8, _N5), lambda b: (0, 0, 0)),
            pl.BlockSpec((1, _C, _N3), bmap),
            pl.BlockSpec((1, _C, _N4), bmap),
            pl.BlockSpec((1, _C, _N5), bmap),
            pl.BlockSpec((1, _NGT), fixed),
            pl.BlockSpec((1, _NGT), fixed),
            pl.BlockSpec((1, _NGT), fixed),
            pl.BlockSpec((1, _NGT), fixed),
            pl.BlockSpec((1, _NGT), fixed),
            pl.BlockSpec((1, _NGT), fixed),
        ],
        out_specs=pl.BlockSpec((1, 128), lambda b: (0, 0)),
        out_shape=jax.ShapeDtypeStruct((1, 128), jnp.float32),
        scratch_shapes=[pltpu.VMEM((_B, _NTOT), jnp.float32)],
        compiler_params=pltpu.CompilerParams(
            dimension_semantics=("arbitrary",)),
    )(p3f, p4f, p5f, p3f, p4f, p5f, gtx, gty, gtw, gth, bidx, gcls)

    lb = out[0, 0] / _B
    lo = out[0, 1] / _B
    lc = out[0, 2] / _B
    total = 0.05 * lb + 1.0 * lo + 0.5 * lc
    return (total, lb, lo, lc)


# speedup vs baseline: 1.2899x; 1.0244x over previous
import jax
import jax.numpy as jnp
from jax.experimental import pallas as pl
from jax.experimental.pallas import tpu as pltpu

_B = 16
_C = 85
_N3, _N4, _N5 = 4096, 1024, 256
_NTOT = _N3 + _N4 + _N5
_R, _L = 42, 128
_K = 50
_NGT = 200
_EPS = 1e-7

_ATAN_C = (0.9999999581953061, -0.3333230282771013, 0.19973681363449028,
           -0.14040138891201454, 0.09967923618944668, -0.060219127990167355,
           0.024756780690475755, -0.00483116838738874)
_HALF_PI = 1.5707963267948966


def _atan(x):
    r = jnp.abs(x)
    z = jnp.minimum(r, 1.0 / r)
    t = z * z
    p = jnp.float32(_ATAN_C[7])
    for c in _ATAN_C[6::-1]:
        p = p * t + jnp.float32(c)
    p = z * p
    res = jnp.where(r <= 1.0, p, _HALF_PI - p)
    return jnp.where(x < 0, -res, res)


def _topk_ranks(o3_ref, o4_ref, o5_ref):
    obj = jnp.concatenate(
        [o3_ref[:, 4, :].reshape(_B, _N3 // _L, _L),
         o4_ref[:, 4, :].reshape(_B, _N4 // _L, _L),
         o5_ref[:, 4, :].reshape(_B, _N5 // _L, _L)], axis=1)
    bits = jax.lax.bitcast_convert_type(obj, jnp.int32)
    skey = bits ^ (jax.lax.shift_right_arithmetic(bits, 31)
                   & jnp.int32(0x7FFFFFFF))

    def bs_body(i, lohi):
        lo, hi = lohi
        mid = (lo >> 1) + (hi >> 1) + ((lo | hi) & 1)
        ge = (skey >= mid).astype(jnp.float32)
        cnt = jnp.sum(jnp.sum(ge, axis=1, keepdims=True),
                      axis=2, keepdims=True)
        take = cnt >= float(_K)
        return jnp.where(take, mid, lo), jnp.where(take, hi, mid - 1)

    lo0 = jnp.full((_B, 1, 1), jnp.int32(-2147483648))
    hi0 = jnp.full((_B, 1, 1), jnp.int32(2147483647))
    thr, _ = jax.lax.fori_loop(0, 32, bs_body, (lo0, hi0), unroll=False)

    lt_l = (jax.lax.broadcasted_iota(jnp.int32, (_L, _L), 0)
            < jax.lax.broadcasted_iota(jnp.int32, (_L, _L), 1)).astype(jnp.float32)
    lt_r = (jax.lax.broadcasted_iota(jnp.int32, (_R, _R), 0)
            < jax.lax.broadcasted_iota(jnp.int32, (_R, _R), 1)).astype(jnp.float32)
    dn2 = (((1,), (0,)), ((), ()))

    def eprefix(mf):
        lane_pref = jax.lax.dot_general(
            mf.reshape(_B * _R, _L), lt_l, dn2,
            preferred_element_type=jnp.float32).reshape(_B, _R, _L)
        rowsum = jnp.sum(mf, axis=2)
        roff = jax.lax.dot_general(rowsum, lt_r, dn2,
                                   preferred_element_type=jnp.float32)
        return lane_pref + roff.reshape(_B, _R, 1)

    strict = (skey > thr).astype(jnp.float32)
    ties = (skey == thr).astype(jnp.float32)
    n1 = jnp.sum(jnp.sum(strict, axis=1, keepdims=True),
                 axis=2, keepdims=True)
    msel = jnp.maximum(strict,
                       ties * (eprefix(ties) < (float(_K) - n1)))
    cc = jnp.where(msel > 0.0, eprefix(msel), -1.0)
    return cc.reshape(_B, _NTOT)


def _loss_kernel(o3_ref, o4_ref, o5_ref, p3_ref, p4_ref, p5_ref,
                 gt_ref, bidx_ref, gcls_ref, out_ref, cc_scr):
    b = pl.program_id(0)

    @pl.when(b == 0)
    def _init():
        out_ref[...] = jnp.zeros_like(out_ref)
        cc_scr[...] = _topk_ranks(o3_ref, o4_ref, o5_ref)

    cc_flat = cc_scr[pl.ds(b, 1), :]
    kcol = jax.lax.broadcasted_iota(jnp.int32, (_K, 1), 0).astype(jnp.float32)
    s = (cc_flat == kcol).astype(jnp.float32)
    dn = (((1,), (1,)), ((), ()))
    sel = (
        jax.lax.dot_general(s[:, :_N3], p3_ref[0], dn,
                            preferred_element_type=jnp.float32)
        + jax.lax.dot_general(s[:, _N3:_N3 + _N4], p4_ref[0], dn,
                              preferred_element_type=jnp.float32)
        + jax.lax.dot_general(s[:, _N3 + _N4:], p5_ref[0], dn,
                              preferred_element_type=jnp.float32)
    )

    ch = jax.lax.broadcasted_iota(jnp.int32, (1, _C), 1)

    def pick(c):
        return jnp.sum(jnp.where(ch == c, sel, 0.0), axis=1, keepdims=True)

    b1x, b1y, b1w, b1h, so = pick(0), pick(1), pick(2), pick(3), pick(4)

    gx = gt_ref[0:1, :]
    gy = gt_ref[1:2, :]
    gw = gt_ref[2:3, :]
    gh = gt_ref[3:4, :]

    b1x1, b1y1 = b1x - b1w * 0.5, b1y - b1h * 0.5
    b1x2, b1y2 = b1x + b1w * 0.5, b1y + b1h * 0.5
    b2x1, b2y1 = gx - gw * 0.5, gy - gh * 0.5
    b2x2, b2y2 = gx + gw * 0.5, gy + gh * 0.5
    iw = jnp.clip(jnp.minimum(b1x2, b2x2) - jnp.maximum(b1x1, b2x1), 0.0, None)
    ih = jnp.clip(jnp.minimum(b1y2, b2y2) - jnp.maximum(b1y1, b2y1), 0.0, None)
    inter = iw * ih
    union = b1w * b1h + gw * gh - inter + _EPS
    iou = inter / union
    cw = jnp.maximum(b1x2, b2x2) - jnp.minimum(b1x1, b2x1)
    chh = jnp.maximum(b1y2, b2y2) - jnp.minimum(b1y1, b2y1)
    c2 = cw * cw + chh * chh + _EPS
    rho2 = (gx - b1x) ** 2 + (gy - b1y) ** 2
    at1 = _atan(b1w / b1h)
    at2 = _atan(gw / gh)
    v = (4.0 / (3.141592653589793 ** 2)) * (at2 - at1) ** 2
    alpha = v / (1.0 - iou + v + _EPS)
    ciou = jnp.clip(iou - (rho2 / c2 + v * alpha), 0.0, 1.0)

    bidx = bidx_ref[...]
    mask = bidx == b
    cm = jnp.where(mask, ciou, -1.0)
    cmax = jnp.max(cm, axis=1, keepdims=True)
    jiota = jax.lax.broadcasted_iota(jnp.int32, (1, _NGT), 1)
    eqm = cm == cmax
    midx = jnp.min(jnp.where(eqm, jiota, _NGT), axis=1, keepdims=True)
    gsel = jnp.sum(jnp.where(jiota == midx, gcls_ref[...], 0.0),
                   axis=1, keepdims=True)

    box_loss = jnp.mean(1.0 - cmax)

    def bce(x, t):
        return jnp.maximum(x, 0.0) - x * t + jnp.log1p(jnp.exp(-jnp.abs(x)))

    obj_loss = jnp.mean(bce(so, cmax))

    chf = ch.astype(jnp.float32)
    cls_mask = ch >= 5
    tgt = jnp.where((chf - 5.0) == gsel, 1.0, 0.0)
    fcls = bce(sel, tgt)
    cls_loss = jnp.sum(jnp.where(cls_mask, fcls, 0.0)) / (_K * (_C - 5))

    has_any = jnp.any(mask)
    l128 = jax.lax.broadcasted_iota(jnp.int32, (1, 128), 1)
    vec = jnp.where(l128 == 0, box_loss,
                    jnp.where(l128 == 1, obj_loss,
                              jnp.where(l128 == 2, cls_loss, 0.0)))
    out_ref[...] += jnp.where(has_any, vec, 0.0)


def kernel(p3, p4, p5, bboxes, cls, batch_idx):
    p3f = p3.reshape(_B, _C, _N3)
    p4f = p4.reshape(_B, _C, _N4)
    p5f = p5.reshape(_B, _C, _N5)
    gt_t = bboxes.T
    bidx = batch_idx.astype(jnp.int32).reshape(1, _NGT)
    gcls = cls[:, 0].astype(jnp.float32).reshape(1, _NGT)

    def bmap(b):
        return (b, 0, 0)

    def fixed(b):
        return (0, 0)

    out = pl.pallas_call(
        _loss_kernel,
        grid=(_B,),
        in_specs=[
            pl.BlockSpec((_B, 8, _N3), lambda b: (0, 0, 0)),
            pl.BlockSpec((_B, 8, _N4), lambda b: (0, 0, 0)),
            pl.BlockSpec((_B, 8, _N5), lambda b: (0, 0, 0)),
            pl.BlockSpec((1, _C, _N3), bmap),
            pl.BlockSpec((1, _C, _N4), bmap),
            pl.BlockSpec((1, _C, _N5), bmap),
            pl.BlockSpec((4, _NGT), fixed),
            pl.BlockSpec((1, _NGT), fixed),
            pl.BlockSpec((1, _NGT), fixed),
        ],
        out_specs=pl.BlockSpec((1, 128), lambda b: (0, 0)),
        out_shape=jax.ShapeDtypeStruct((1, 128), jnp.float32),
        scratch_shapes=[pltpu.VMEM((_B, _NTOT), jnp.float32)],
        compiler_params=pltpu.CompilerParams(
            dimension_semantics=("arbitrary",)),
    )(p3f, p4f, p5f, p3f, p4f, p5f, gt_t, bidx, gcls)

    lb = out[0, 0] / _B
    lo = out[0, 1] / _B
    lc = out[0, 2] / _B
    total = 0.05 * lb + 1.0 * lo + 0.5 * lc
    return (total, lb, lo, lc)
